# Initial kernel scaffold; baseline (speedup 1.0000x reference)
import jax
import jax.numpy as jnp
from jax.experimental import pallas as pl


def kernel(edge_index, A_vals, alpha, edgenet, pnet):
    raise NotImplementedError("write your pallas kernel here")


# speedup vs baseline: 593.4704x; 593.4704x over previous
import functools

import jax
import jax.numpy as jnp
import numpy as np
from jax import lax
from jax.experimental import pallas as pl
from jax.experimental.pallas import tpu as pltpu
from jax.experimental.pallas import tpu_sc as plsc

N_NODES = 10000
N_EDGES = 160000
DIM = 64
N_CONV = 4

_CH = 20000
_NCHUNK = N_EDGES // _CH
_NB = N_NODES // 16


def _mpnn(params, edge_index, edge_feats, n):
    src, dst = edge_index[0], edge_index[1]
    e = jax.nn.relu(edge_feats @ params["We"])
    x = jax.nn.relu(jnp.ones((n, 1), jnp.float32) @ params["Wx"])
    for l in range(N_CONV):
        msg_in = jnp.concatenate([x[src], x[dst], e], axis=1)
        m = jax.nn.relu(msg_in @ params["Wm"][l] + params["bm"][l])
        agg = jnp.zeros((n, m.shape[1]), jnp.float32).at[dst].add(m)
        x = jax.nn.relu(jnp.concatenate([x, agg], axis=1) @ params["Wu"][l] + params["bu"][l])
        e = jax.nn.relu(msg_in @ params["Wem"][l] + params["bem"][l])
    return x, e @ params["Wo"]


def _splat(x):
    return jnp.full((16,), x, jnp.int32)


def _partition_body(src_hbm, dst_hbm, k_hbm, cols_hbm,
                    parent_v, colmap_v, colsout_v, srcb, dstb, kb):
    on0 = (lax.axis_index("c") == 0) & (lax.axis_index("s") == 0)

    def g1(ref, x):
        return plsc.load_gather(ref, [_splat(x)])[0]

    def s1(ref, i, val):
        plsc.store_scatter(ref, [_splat(i)], _splat(val))

    @pl.when(on0)
    def _():
        pltpu.sync_copy(k_hbm, kb)
        k = kb[...][0]

        iota = lax.iota(jnp.int32, 16)

        def init_blk(b, u):
            parent_v[pl.ds(b * 16, 16)] = iota + (b * 16)
            return u

        lax.fori_loop(0, _NB, init_blk, jnp.int32(0))

        def find(u):
            def cond(c):
                r, pr = c
                return pr != r

            def body(c):
                r, pr = c
                gp = g1(parent_v, pr)
                s1(parent_v, r, gp)
                return pr, gp

            r, _ = lax.while_loop(cond, body, (u, g1(parent_v, u)))
            return r

        num_sets = jnp.int32(N_NODES)
        for c in range(_NCHUNK):
            pltpu.sync_copy(src_hbm.at[pl.ds(c * _CH, _CH)], srcb)
            pltpu.sync_copy(dst_hbm.at[pl.ds(c * _CH, _CH)], dstb)

            def blk_cond(carry):
                jb, ns = carry
                return (jb < _CH // 16) & (ns > k)

            def blk_body(carry):
                jb, ns = carry
                u16 = srcb[pl.ds(jb * 16, 16)]
                v16 = dstb[pl.ds(jb * 16, 16)]
                for i in range(16):
                    ru = find(u16[i])
                    rv = find(v16[i])
                    do = (ns > k) & (ru != rv)

                    @pl.when(do)
                    def _():
                        s1(parent_v, ru, rv)

                    ns = ns - do.astype(jnp.int32)
                return jb + 1, ns

            _, num_sets = lax.while_loop(
                blk_cond, blk_body, (jnp.int32(0), num_sets))

        def jump_pass(_, unused):
            def blk(b, u):
                idx = parent_v[pl.ds(b * 16, 16)]
                parent_v[pl.ds(b * 16, 16)] = plsc.load_gather(parent_v, [idx])
                return u

            return lax.fori_loop(0, _NB, blk, unused)

        lax.fori_loop(0, 15, jump_pass, jnp.int32(0))

        zeros16 = jnp.zeros((16,), jnp.int32)
        ones16 = jnp.ones((16,), jnp.int32)

        def zero_blk(b, u):
            colmap_v[pl.ds(b * 16, 16)] = zeros16
            return u

        lax.fori_loop(0, _NB, zero_blk, jnp.int32(0))

        def mark_blk(b, u):
            r = parent_v[pl.ds(b * 16, 16)]
            plsc.store_scatter(colmap_v, [r], ones16)
            return u

        lax.fori_loop(0, _NB, mark_blk, jnp.int32(0))

        def scan_blk(b, s):
            v = colmap_v[pl.ds(b * 16, 16)]
            c = plsc.cumsum(v)
            colmap_v[pl.ds(b * 16, 16)] = c + (s - 1)
            return s + jnp.sum(v)

        lax.fori_loop(0, _NB, scan_blk, jnp.int32(0))

        def col_blk(b, u):
            r = parent_v[pl.ds(b * 16, 16)]
            colsout_v[pl.ds(b * 16, 16)] = plsc.load_gather(colmap_v, [r])
            return u

        lax.fori_loop(0, _NB, col_blk, jnp.int32(0))
        pltpu.sync_copy(colsout_v, cols_hbm)


def _partition_sc(src_sorted, dst_sorted, k):
    mesh = plsc.VectorSubcoreMesh(core_axis_name="c", subcore_axis_name="s")
    kern = functools.partial(
        pl.kernel,
        mesh=mesh,
        compiler_params=pltpu.CompilerParams(needs_layout_passes=False),
        out_type=jax.ShapeDtypeStruct((N_NODES,), jnp.int32),
        scratch_types=[
            pltpu.VMEM((N_NODES,), jnp.int32),
            pltpu.VMEM((N_NODES,), jnp.int32),
            pltpu.VMEM((N_NODES,), jnp.int32),
            pltpu.VMEM((_CH,), jnp.int32),
            pltpu.VMEM((_CH,), jnp.int32),
            pltpu.VMEM((16,), jnp.int32),
        ],
    )(_partition_body)
    k_arr = jnp.full((16,), k, jnp.int32)
    return kern(src_sorted, dst_sorted, k_arr)


def _partition(edge_index, a, alpha, n):
    order = jnp.argsort(-a)
    src = edge_index[0][order].astype(jnp.int32)
    dst = edge_index[1][order].astype(jnp.int32)
    kf = jnp.maximum(jnp.ceil(alpha.reshape(-1)[0].astype(jnp.float64) * n), 1.0)
    k = kf.astype(jnp.int32)
    return _partition_sc(src, dst, k).astype(edge_index.dtype)


def _same_kernel(cs_ref, cd_ref, out_ref):
    out_ref[...] = (cs_ref[...] == cd_ref[...]).astype(jnp.float32)


def _build_same(cols_src, cols_dst):
    rows = N_EDGES // 128
    out = pl.pallas_call(
        _same_kernel,
        out_shape=jax.ShapeDtypeStruct((rows, 128), jnp.float32),
    )(cols_src.reshape(rows, 128), cols_dst.reshape(rows, 128))
    return out.reshape(N_EDGES)


def kernel(edge_index, A_vals, alpha, edgenet, pnet):
    n = N_NODES
    _, a_e = _mpnn(edgenet, edge_index, A_vals[:, None], n)
    a = a_e.squeeze(-1)
    cols = _partition(edge_index, a, alpha, n)
    same = _build_same(cols[edge_index[0]], cols[edge_index[1]])
    ef2 = jnp.stack([A_vals, same], axis=1)
    _, p_e = _mpnn(pnet, edge_index, ef2, n)
    p = p_e.squeeze(-1)
    P_idx = jnp.stack([edge_index[0], cols[edge_index[1]]])
    return cols, P_idx, jnp.stack([p, a])
